# baseline (speedup 1.0000x reference)
import functools

import jax
import jax.numpy as jnp
from jax import lax
from jax.experimental import pallas as pl
from jax.experimental.pallas import tpu as pltpu
from jax.experimental.pallas import tpu_sc as plsc

NC, NS, L = 2, 16, 16
NW = NC * NS
B = 16384
D = 32
NCTX = 5
BPW = B // NW
JPW = BPW * NCTX
CHUNK = 64
NT_CH = BPW // CHUNK
NCT_CH = JPW // CHUNK
NGRP = BPW // L


def _w2v_body(tgt_hbm, ctx_hbm, wt_hbm, wc_hbm, out_hbm,
              tgt_idx, ctx_idx, we, ce, out_v, sem):
    wid = lax.axis_index("s") * NC + lax.axis_index("c")

    pltpu.sync_copy(tgt_hbm.at[pl.ds(wid * NT_CH, NT_CH)], tgt_idx)
    pltpu.sync_copy(ctx_hbm.at[pl.ds(wid * NCT_CH, NCT_CH)], ctx_idx)

    copies = []
    for c in range(NT_CH):
        copies.append(pltpu.async_copy(
            wt_hbm.at[tgt_idx.at[c]], we.at[pl.ds(c * CHUNK, CHUNK)], sem))
    for c in range(NCT_CH):
        copies.append(pltpu.async_copy(
            wc_hbm.at[ctx_idx.at[c]], ce.at[pl.ds(c * CHUNK, CHUNK)], sem))
    for cp in copies:
        cp.wait()

    iota = lax.iota(jnp.int32, L)

    @pl.loop(0, NGRP)
    def _group(g):
        b_row = iota + g * L
        wecols = [
            plsc.load_gather(we, [b_row, jnp.full((L,), d, jnp.int32)])
            for d in range(D)
        ]
        for n in range(NCTX):
            j_row = iota * NCTX + (g * (L * NCTX) + n)
            acc = wecols[0] * plsc.load_gather(
                ce, [j_row, jnp.full((L,), 0, jnp.int32)])
            for d in range(1, D):
                cev = plsc.load_gather(
                    ce, [j_row, jnp.full((L,), d, jnp.int32)])
                acc = acc + wecols[d] * cev
            plsc.store_scatter(out_v, [j_row], acc)

    pltpu.sync_copy(out_v, out_hbm.at[pl.ds(wid * JPW, JPW)])


@jax.jit
def kernel(target, context, W_target, W_context):
    tgt2d = target.reshape(B // CHUNK, CHUNK)
    ctx2d = context.reshape(B * NCTX // CHUNK, CHUNK)

    mesh = plsc.VectorSubcoreMesh(
        core_axis_name="c", subcore_axis_name="s",
        num_cores=NC, num_subcores=NS)
    out_flat = pl.kernel(
        _w2v_body,
        out_type=jax.ShapeDtypeStruct((B * NCTX,), jnp.float32),
        mesh=mesh,
        compiler_params=pltpu.CompilerParams(
            needs_layout_passes=False, use_tc_tiling_on_sc=False),
        scratch_types=[
            pltpu.VMEM((NT_CH, CHUNK), jnp.int32),
            pltpu.VMEM((NCT_CH, CHUNK), jnp.int32),
            pltpu.VMEM((BPW, D), jnp.float32),
            pltpu.VMEM((JPW, D), jnp.float32),
            pltpu.VMEM((JPW,), jnp.float32),
            pltpu.SemaphoreType.DMA,
        ],
    )(tgt2d, ctx2d, W_target, W_context)
    return out_flat.reshape(B, NCTX)
